# baseline (speedup 1.0000x reference)
import functools

import jax
import jax.numpy as jnp
from jax.experimental import pallas as pl
from jax.experimental.pallas import tpu as pltpu

_ECHUNK = 2048


def _bf16(a):
    return a.astype(jnp.bfloat16)


def _dot(a, b):
    return jnp.dot(a, b, preferred_element_type=jnp.float32)


def _dot_tb(a, b):
    return jax.lax.dot_general(a, b, (((1,), (1,)), ((), ())),
                               preferred_element_type=jnp.float32)


def _fused_kernel(ei_ref, v_ref, x_ref, na_ref, ew_ref, eb_ref,
                  w0_ref, b0_ref, w1_ref, b1_ref, w2_ref, b2_ref,
                  w3_ref, b3_ref, w4_ref, b4_ref, w5_ref, b5_ref,
                  o1w_ref, o1b_ref, o2w_ref, o2b_ref,
                  o_ref, hbuf,
                  *, nchunks, n, hidden, a_dim):
    e = ei_ref.shape[1]
    ec = e // nchunks

    accf = None
    sacc = None
    for k in range(nchunks):
        sl = slice(k * ec, (k + 1) * ec)
        srcv = ei_ref[0:1, sl]
        dstv = ei_ref[1:2, sl]
        vals = _bf16(v_ref[:, sl])
        eav = vals[a_dim + 1:, :]
        ut = _bf16(jax.lax.broadcasted_iota(jnp.int32, (n, ec), 0) == dstv)
        st = _bf16(jax.lax.broadcasted_iota(jnp.int32, (n, ec), 0) == srcv)
        pa = _dot_tb(ut * eav, st)
        ps = _dot_tb(ut, vals)
        accf = pa if accf is None else accf + pa
        sacc = ps if sacc is None else sacc + ps

    adj16 = _bf16(accf)
    ce16 = _bf16(sacc[:, :a_dim])
    cnt = sacc[:, a_dim:a_dim + 1]
    deg = sacc[:, a_dim + 1:a_dim + 2]
    na = na_ref[...].reshape(n, 1)

    hbuf[...] = (_dot(_bf16(x_ref[...] * na), _bf16(ew_ref[...]))
                 + eb_ref[...])

    for w_ref, b_ref in ((w0_ref, b0_ref), (w1_ref, b1_ref),
                         (w2_ref, b2_ref), (w3_ref, b3_ref),
                         (w4_ref, b4_ref), (w5_ref, b5_ref)):
        h = hbuf[...]
        h16 = _bf16(h)
        ah = _dot(adj16, h16)
        agg = (
            deg * _dot(h16, _bf16(w_ref[:hidden]))
            + _dot(_bf16(ah), _bf16(w_ref[hidden:2 * hidden]))
            + _dot(ce16, _bf16(w_ref[2 * hidden:2 * hidden + a_dim]))
            + cnt * b_ref[...]
        )
        hbuf[...] = h + agg * jax.nn.sigmoid(agg)

    h = hbuf[...]
    t = _dot(_bf16(h * na), _bf16(o1w_ref[...])) + o1b_ref[...]
    t = t * jax.nn.sigmoid(t)
    o_ref[...] = _dot(_bf16(t * na), _bf16(o2w_ref[...])) + o2b_ref[...]


@jax.jit
def kernel(x, edge_index, amf, node_attr, edge_attr, embed_w, embed_b,
           out1_w, out1_b, out2_w, out2_b, layer0_w, layer0_b, layer1_w,
           layer1_b, layer2_w, layer2_b, layer3_w, layer3_b, layer4_w,
           layer4_b, layer5_w, layer5_b):
    n, in_dim = x.shape
    e = edge_index.shape[1]
    hidden = embed_w.shape[1]
    out_dim = out2_w.shape[1]
    a_dim = amf.shape[1]
    wrows = layer0_w.shape[0]
    ec = min(_ECHUNK, e)
    nchunks = e // ec

    ea_row = edge_attr.T
    v = jnp.concatenate(
        [amf.T * ea_row, jnp.ones((1, e), jnp.float32), ea_row], axis=0)

    def _const(shape):
        zeros = (0,) * len(shape)
        return pl.BlockSpec(shape, lambda k, _z=zeros: _z)

    lspecs = []
    for _ in range(6):
        lspecs += [_const((wrows, hidden)), _const((1, hidden))]

    out = pl.pallas_call(
        functools.partial(_fused_kernel, nchunks=nchunks, n=n,
                          hidden=hidden, a_dim=a_dim),
        out_shape=jax.ShapeDtypeStruct((n, out_dim), jnp.float32),
        grid=(1,),
        in_specs=[
            _const((2, e)),
            _const((a_dim + 2, e)),
            _const((n, in_dim)),
            _const((1, n)),
            _const((in_dim, hidden)),
            _const((1, hidden)),
        ] + lspecs + [
            _const((hidden, hidden)),
            _const((1, hidden)),
            _const((hidden, out_dim)),
            _const((1, out_dim)),
        ],
        out_specs=pl.BlockSpec((n, out_dim), lambda k: (0, 0)),
        scratch_shapes=[
            pltpu.VMEM((n, hidden), jnp.float32),
        ],
        compiler_params=pltpu.CompilerParams(
            dimension_semantics=("arbitrary",),
            vmem_limit_bytes=58 << 20),
    )(edge_index, v, x, node_attr.T, embed_w, embed_b,
      layer0_w, layer0_b, layer1_w, layer1_b, layer2_w, layer2_b,
      layer3_w, layer3_b, layer4_w, layer4_b, layer5_w, layer5_b,
      out1_w, out1_b, out2_w, out2_b)

    return out


# speedup vs baseline: 1.1676x; 1.1676x over previous
import functools

import jax
import jax.numpy as jnp
from jax.experimental import pallas as pl
from jax.experimental.pallas import tpu as pltpu

_ECHUNK = 8192


def _bf16(a):
    return a.astype(jnp.bfloat16)


def _dot(a, b):
    return jnp.dot(a, b, preferred_element_type=jnp.float32)


def _dot_tb(a, b):
    return jax.lax.dot_general(a, b, (((1,), (1,)), ((), ())),
                               preferred_element_type=jnp.float32)


def _fused_kernel(ei_ref, v_ref, x_ref, na_ref, ew_ref, eb_ref,
                  w0_ref, b0_ref, w1_ref, b1_ref, w2_ref, b2_ref,
                  w3_ref, b3_ref, w4_ref, b4_ref, w5_ref, b5_ref,
                  o1w_ref, o1b_ref, o2w_ref, o2b_ref,
                  o_ref, accf, sacc, adj16, hbuf,
                  *, nchunks, n, hidden, a_dim):
    k = pl.program_id(0)
    ec = ei_ref.shape[1]

    @pl.when(k == 0)
    def _():
        accf[...] = jnp.zeros_like(accf)
        sacc[...] = jnp.zeros_like(sacc)

    srcv = ei_ref[0:1, :]
    dstv = ei_ref[1:2, :]
    vals = _bf16(v_ref[...])
    eav = vals[a_dim + 1:, :]
    ut = _bf16(jax.lax.broadcasted_iota(jnp.int32, (n, ec), 0) == dstv)
    st = _bf16(jax.lax.broadcasted_iota(jnp.int32, (n, ec), 0) == srcv)
    accf[...] += _dot_tb(ut * eav, st)
    sacc[...] += _dot_tb(ut, vals)

    @pl.when(k == nchunks - 1)
    def _():
        adj16[...] = _bf16(accf[...])
        ce16 = _bf16(sacc[:, :a_dim])
        cnt = sacc[:, a_dim:a_dim + 1]
        deg = sacc[:, a_dim + 1:a_dim + 2]
        na = na_ref[...].reshape(n, 1)

        hbuf[...] = (_dot(_bf16(x_ref[...] * na), _bf16(ew_ref[...]))
                     + eb_ref[...])

        for w_ref, b_ref in ((w0_ref, b0_ref), (w1_ref, b1_ref),
                             (w2_ref, b2_ref), (w3_ref, b3_ref),
                             (w4_ref, b4_ref), (w5_ref, b5_ref)):
            h = hbuf[...]
            h16 = _bf16(h)
            ah = _dot(adj16[...], h16)
            agg = (
                deg * _dot(h16, _bf16(w_ref[:hidden]))
                + _dot(_bf16(ah), _bf16(w_ref[hidden:2 * hidden]))
                + _dot(ce16, _bf16(w_ref[2 * hidden:2 * hidden + a_dim]))
                + cnt * b_ref[...]
            )
            hbuf[...] = h + agg * jax.nn.sigmoid(agg)

        h = hbuf[...]
        t = _dot(_bf16(h * na), _bf16(o1w_ref[...])) + o1b_ref[...]
        t = t * jax.nn.sigmoid(t)
        o_ref[...] = _dot(_bf16(t * na), _bf16(o2w_ref[...])) + o2b_ref[...]


@jax.jit
def kernel(x, edge_index, amf, node_attr, edge_attr, embed_w, embed_b,
           out1_w, out1_b, out2_w, out2_b, layer0_w, layer0_b, layer1_w,
           layer1_b, layer2_w, layer2_b, layer3_w, layer3_b, layer4_w,
           layer4_b, layer5_w, layer5_b):
    n, in_dim = x.shape
    e = edge_index.shape[1]
    hidden = embed_w.shape[1]
    out_dim = out2_w.shape[1]
    a_dim = amf.shape[1]
    wrows = layer0_w.shape[0]
    ec = min(_ECHUNK, e)
    nchunks = e // ec

    ea_row = edge_attr.T
    v = jnp.concatenate(
        [amf.T * ea_row, jnp.ones((1, e), jnp.float32), ea_row], axis=0)

    def _const(shape):
        zeros = (0,) * len(shape)
        return pl.BlockSpec(shape, lambda k, _z=zeros: _z)

    lspecs = []
    for _ in range(6):
        lspecs += [_const((wrows, hidden)), _const((1, hidden))]

    out = pl.pallas_call(
        functools.partial(_fused_kernel, nchunks=nchunks, n=n,
                          hidden=hidden, a_dim=a_dim),
        out_shape=jax.ShapeDtypeStruct((n, out_dim), jnp.float32),
        grid=(nchunks,),
        in_specs=[
            pl.BlockSpec((2, ec), lambda k: (0, k)),
            pl.BlockSpec((a_dim + 2, ec), lambda k: (0, k)),
            _const((n, in_dim)),
            _const((1, n)),
            _const((in_dim, hidden)),
            _const((1, hidden)),
        ] + lspecs + [
            _const((hidden, hidden)),
            _const((1, hidden)),
            _const((hidden, out_dim)),
            _const((1, out_dim)),
        ],
        out_specs=pl.BlockSpec((n, out_dim), lambda k: (0, 0)),
        scratch_shapes=[
            pltpu.VMEM((n, n), jnp.float32),
            pltpu.VMEM((n, a_dim + 2), jnp.float32),
            pltpu.VMEM((n, n), jnp.bfloat16),
            pltpu.VMEM((n, hidden), jnp.float32),
        ],
        compiler_params=pltpu.CompilerParams(
            dimension_semantics=("arbitrary",),
            vmem_limit_bytes=58 << 20),
    )(edge_index, v, x, node_attr.T, embed_w, embed_b,
      layer0_w, layer0_b, layer1_w, layer1_b, layer2_w, layer2_b,
      layer3_w, layer3_b, layer4_w, layer4_b, layer5_w, layer5_b,
      out1_w, out1_b, out2_w, out2_b)

    return out
